# baseline (speedup 1.0000x reference)
import functools

import jax
import jax.numpy as jnp
from jax import lax
from jax.experimental import pallas as pl
from jax.experimental.pallas import tpu as pltpu
from jax.experimental.pallas import tpu_sc as plsc

LANES = 16
NWORKERS = 32
NBUF = 2
CHUNK = 128
DIM = 64
PDIM = 2 * DIM
SEQP = 2 * CHUNK


def _sc_pool(idx2_flat, off_flat, bsz, seq, emb2, cat0p, cat1p, cat2p,
             cidx2_flat, coff_flat):
  bpw = bsz // NWORKERS
  ngrp = DIM // LANES
  nblk = seq // LANES
  tail = seq - nblk * LANES
  mesh = plsc.VectorSubcoreMesh(core_axis_name="c", subcore_axis_name="s")

  @functools.partial(
      pl.kernel,
      out_type=(
          jax.ShapeDtypeStruct((bsz // 2, PDIM), jnp.float32),
          jax.ShapeDtypeStruct((bsz // 2, PDIM), jnp.float32),
      ),
      mesh=mesh,
      scratch_types=[
          pltpu.VMEM((NBUF * SEQP,), jnp.int32),
          pltpu.VMEM((NBUF * SEQP,), jnp.int32),
          pltpu.VMEM((NBUF, seq, PDIM), jnp.float32),
          pltpu.VMEM((bpw // 2, PDIM), jnp.float32),
          pltpu.VMEM((bpw // 2, PDIM), jnp.float32),
          pltpu.VMEM((bpw, PDIM), jnp.float32),
          pltpu.VMEM((bpw, PDIM), jnp.float32),
          pltpu.VMEM((3 * bpw,), jnp.int32),
          pltpu.VMEM((3 * bpw,), jnp.int32),
          pltpu.SemaphoreType.DMA,
          pltpu.SemaphoreType.DMA,
      ],
  )
  def k(idx2_hbm, off_hbm, emb_hbm, c0_hbm, c1_hbm, c2_hbm,
        cidx_hbm, coff_hbm,
        sums_hbm, cats_hbm,
        idx_v, off_v, rows_v, acc_v, cacc_v, ca_v, cb_v, cidx_v, coff_v,
        sem0, sem1):
    sems = (sem0, sem1)
    wid = lax.axis_index("s") * 2 + lax.axis_index("c")
    base = wid * bpw

    def gather_descs(buf):
      return (
          pltpu.make_async_copy(
              emb_hbm.at[idx_v.at[pl.ds(buf * SEQP, CHUNK)]],
              rows_v.at[buf, pl.ds(0, CHUNK)], sems[buf]),
          pltpu.make_async_copy(
              emb_hbm.at[idx_v.at[pl.ds(buf * SEQP + CHUNK, seq - CHUNK)]],
              rows_v.at[buf, pl.ds(CHUNK, seq - CHUNK)], sems[buf]),
      )

    def issue(buf, row):
      for c in range(2):
        src = pl.multiple_of(row * SEQP + c * CHUNK, CHUNK)
        pltpu.sync_copy(idx2_hbm.at[pl.ds(src, CHUNK)],
                        idx_v.at[pl.ds(buf * SEQP + c * CHUNK, CHUNK)])
        pltpu.sync_copy(off_hbm.at[pl.ds(src, CHUNK)],
                        off_v.at[pl.ds(buf * SEQP + c * CHUNK, CHUNK)])
      for d in gather_descs(buf):
        d.start()

    for buf in range(NBUF):
      issue(buf, base + buf)

    for j in range(3):
      csrc = pl.multiple_of(j * bsz + base, CHUNK)
      pltpu.sync_copy(cidx_hbm.at[pl.ds(csrc, bpw)],
                      cidx_v.at[pl.ds(j * bpw, bpw)])
      pltpu.sync_copy(coff_hbm.at[pl.ds(csrc, bpw)],
                      coff_v.at[pl.ds(j * bpw, bpw)])
    pltpu.sync_copy(c0_hbm.at[cidx_v.at[pl.ds(0, bpw)]], ca_v)
    pltpu.sync_copy(c1_hbm.at[cidx_v.at[pl.ds(bpw, bpw)]], cb_v)

    def cat_accum(b0, src_v, joff, first):
      offs = coff_v[pl.ds(joff * bpw + b0, LANES)]
      for kk in range(LANES):
        off = offs[kk]
        row = (b0 >> 1) + (kk >> 1)
        col = (kk & 1) * DIM
        for j in range(ngrp):
          sl = pl.ds(col + j * LANES, LANES)
          v = src_v[b0 + kk, pl.ds(off + j * LANES, LANES)]
          if first:
            cacc_v[row, sl] = v
          else:
            cacc_v[row, sl] = cacc_v[row, sl] + v

    @pl.loop(0, bpw, step=LANES)
    def _(b0):
      cat_accum(b0, ca_v, 0, True)
      cat_accum(b0, cb_v, 1, False)

    pltpu.sync_copy(c2_hbm.at[cidx_v.at[pl.ds(2 * bpw, bpw)]], ca_v)

    @pl.loop(0, bpw, step=LANES)
    def _(b0):
      cat_accum(b0, ca_v, 2, False)

    pltpu.sync_copy(cacc_v,
                    cats_hbm.at[pl.ds(pl.multiple_of(base // 2, 8), bpw // 2)])

    @pl.loop(0, bpw // NBUF)
    def _(i):
      for buf in range(NBUF):
        b_local = i * NBUF + buf
        for d in gather_descs(buf):
          d.wait()
        zeros = (jnp.zeros((LANES,), jnp.float32),) * ngrp

        def tok_block(t0, carry, cnt):
          offs = off_v[pl.ds(buf * SEQP + t0, LANES)]
          acc = list(carry)
          for kk in range(cnt):
            off = offs[kk]
            for j in range(ngrp):
              acc[j] = acc[j] + rows_v[buf, t0 + kk,
                                       pl.ds(off + j * LANES, LANES)]
          return tuple(acc)

        @pl.loop(0, nblk * LANES, step=LANES, init_carry=zeros, unroll=4)
        def totals(t0, carry):
          return tok_block(t0, carry, LANES)

        if tail:
          totals = tok_block(nblk * LANES, totals, tail)

        col = (buf & 1) * DIM
        for j in range(ngrp):
          acc_v[i, pl.ds(col + j * LANES, LANES)] = totals[j]
        @pl.when(i < bpw // NBUF - 1)
        def _():
          issue(buf, base + b_local + NBUF)

    pltpu.sync_copy(acc_v,
                    sums_hbm.at[pl.ds(pl.multiple_of(base // 2, 8), bpw // 2)])

  return k(idx2_flat, off_flat, emb2, cat0p, cat1p, cat2p,
           cidx2_flat, coff_flat)


def _tc_head(sums, cats, encoded_text, w_t, bias):
  bsz, seq = encoded_text.shape
  dim = sums.shape[1]
  ncls = w_t.shape[1]
  blk = 256

  def body(sums_ref, cats_ref, enc_ref, wt_ref, b_ref, out_ref):
    cnt = jnp.sum((enc_ref[...] != 0).astype(jnp.float32), axis=1,
                  keepdims=True)
    x = jnp.where(cnt > 0.0, sums_ref[...] / cnt, 0.0)
    x = x + cats_ref[...]
    z = lax.dot_general(x, wt_ref[...], (((1,), (0,)), ((), ())),
                        preferred_element_type=jnp.float32)
    out_ref[...] = z + b_ref[...]

  return pl.pallas_call(
      body,
      grid=(bsz // blk,),
      in_specs=[
          pl.BlockSpec((blk, dim), lambda i: (i, 0)),
          pl.BlockSpec((blk, dim), lambda i: (i, 0)),
          pl.BlockSpec((blk, seq), lambda i: (i, 0)),
          pl.BlockSpec((dim, ncls), lambda i: (0, 0)),
          pl.BlockSpec((1, ncls), lambda i: (0, 0)),
      ],
      out_specs=pl.BlockSpec((blk, ncls), lambda i: (i, 0)),
      out_shape=jax.ShapeDtypeStruct((bsz, ncls), jnp.float32),
  )(sums, cats, encoded_text, w_t, bias)


def kernel(encoded_text, additional_inputs, emb_table, cat0, cat1, cat2, W, b):
  bsz, seq = encoded_text.shape
  vocab = emb_table.shape[0]
  enc_p = jnp.pad(encoded_text, ((0, 0), (0, SEQP - seq)))
  idx2_flat = jnp.right_shift(enc_p, 1).reshape(-1)
  off_flat = (jnp.bitwise_and(enc_p, 1) * DIM).reshape(-1)
  emb2 = emb_table.reshape(vocab // 2, PDIM)
  cat0p = cat0.reshape(-1, PDIM)
  cat1p = cat1.reshape(-1, PDIM)
  cat2p = cat2.reshape(-1, PDIM)
  add_t = additional_inputs.T
  cidx2_flat = jnp.right_shift(add_t, 1).reshape(-1)
  coff_flat = (jnp.bitwise_and(add_t, 1) * DIM).reshape(-1)
  sums2, cats2 = _sc_pool(idx2_flat, off_flat, bsz, seq, emb2,
                          cat0p, cat1p, cat2p, cidx2_flat, coff_flat)
  sums = sums2.reshape(bsz, DIM)
  cats = cats2.reshape(bsz, DIM)
  return _tc_head(sums, cats, encoded_text, W.T, b.reshape(1, -1))


# speedup vs baseline: 1.0047x; 1.0047x over previous
import functools

import jax
import jax.numpy as jnp
from jax import lax
from jax.experimental import pallas as pl
from jax.experimental.pallas import tpu as pltpu
from jax.experimental.pallas import tpu_sc as plsc

LANES = 16
NWORKERS = 32
NBUF = 2
CHUNK = 128
DIM = 64
PDIM = 2 * DIM
SEQP = 256


def _pack_table(tab):
  n = tab.shape[0]
  h = n // 2

  if h % 1000 == 0:
    blk, grid = 1000, h // 1000

    def body(a_ref, b_ref, o_ref):
      o_ref[:, :DIM] = a_ref[...]
      o_ref[:, DIM:] = b_ref[...]

    return pl.pallas_call(
        body,
        grid=(grid,),
        in_specs=[
            pl.BlockSpec((blk, DIM), lambda i: (i, 0)),
            pl.BlockSpec((blk, DIM), lambda i: (i + grid, 0)),
        ],
        out_specs=pl.BlockSpec((blk, PDIM), lambda i: (i, 0)),
        out_shape=jax.ShapeDtypeStruct((h, PDIM), jnp.float32),
    )(tab, tab)

  def body1(a_ref, o_ref):
    o_ref[:, :DIM] = a_ref[:h, :]
    o_ref[:, DIM:] = a_ref[h:, :]

  return pl.pallas_call(
      body1,
      out_shape=jax.ShapeDtypeStruct((h, PDIM), jnp.float32),
  )(tab)


def _sc_pool(enc_flat, embp, c0p, c1p, c2p, cidx_flat, bsz, seq,
             hemb, hcats):
  bpw = bsz // NWORKERS
  ngrp = DIM // LANES
  nrow = bpw
  mesh = plsc.VectorSubcoreMesh(core_axis_name="c", subcore_axis_name="s")

  @functools.partial(
      pl.kernel,
      out_type=(
          jax.ShapeDtypeStruct((bsz * DIM,), jnp.float32),
          jax.ShapeDtypeStruct((bsz * DIM,), jnp.float32),
      ),
      mesh=mesh,
      scratch_types=[
          pltpu.VMEM((NBUF * SEQP,), jnp.int32),
          pltpu.VMEM((NBUF * SEQP,), jnp.int32),
          pltpu.VMEM((NBUF, seq, PDIM), jnp.float32),
          pltpu.VMEM((bpw * DIM,), jnp.float32),
          pltpu.VMEM((bpw * DIM,), jnp.float32),
          pltpu.VMEM((3 * bpw,), jnp.int32),
          pltpu.VMEM((3 * bpw,), jnp.int32),
          pltpu.VMEM((1, bpw, PDIM), jnp.float32),
          pltpu.SemaphoreType.DMA,
          pltpu.SemaphoreType.DMA,
          pltpu.SemaphoreType.DMA,
          pltpu.SemaphoreType.DMA,
      ],
  )
  def k(enc_hbm, emb_hbm, c0_hbm, c1_hbm, c2_hbm, cidx_hbm,
        sums_hbm, cats_hbm,
        enc_v, idx_v, rows_v, acc_v, cacc_v, cidx_v, cgi_v, cbuf_v,
        se0, se1, sg0, sg1):
    sems_e = (se0, se1)
    sems_g = (sg0, sg1)
    wid = lax.axis_index("s") * 2 + lax.axis_index("c")
    base = wid * nrow

    def enc_desc(slot, row):
      src = pl.multiple_of(row * SEQP, SEQP)
      return pltpu.make_async_copy(
          enc_hbm.at[pl.ds(src, SEQP)],
          enc_v.at[pl.ds(slot * SEQP, SEQP)], sems_e[slot])

    def gather_descs(slot):
      return (
          pltpu.make_async_copy(
              emb_hbm.at[idx_v.at[pl.ds(slot * SEQP, CHUNK)]],
              rows_v.at[slot, pl.ds(0, CHUNK)], sems_g[slot]),
          pltpu.make_async_copy(
              emb_hbm.at[idx_v.at[pl.ds(slot * SEQP + CHUNK, seq - CHUNK)]],
              rows_v.at[slot, pl.ds(CHUNK, seq - CHUNK)], sems_g[slot]),
      )

    def prep_and_fire(slot):
      for t0 in range(0, seq, LANES):
        tv = enc_v[pl.ds(slot * SEQP + t0, LANES)]
        idx_v[pl.ds(slot * SEQP + t0, LANES)] = jnp.where(
            tv >= hemb, tv - hemb, tv)
      for d in gather_descs(slot):
        d.start()

    enc_desc(0, base).start()
    enc_desc(1, base + 1).start()
    enc_desc(0, base).wait()
    prep_and_fire(0)

    csrc = pl.multiple_of(base, CHUNK)
    for j in range(3):
      pltpu.sync_copy(cidx_hbm.at[pl.ds(csrc + j * bsz, bpw)],
                      cidx_v.at[pl.ds(j * bpw, bpw)])
    for j, hc in enumerate(hcats):
      for b0 in range(0, bpw, LANES):
        av = cidx_v[pl.ds(j * bpw + b0, LANES)]
        cgi_v[pl.ds(j * bpw + b0, LANES)] = jnp.where(av >= hc, av - hc, av)

    for j, (tab, hc) in enumerate(zip((c0_hbm, c1_hbm, c2_hbm), hcats)):
      pltpu.sync_copy(tab.at[cgi_v.at[pl.ds(j * bpw, bpw)]], cbuf_v.at[0])

      @pl.loop(0, bpw, step=LANES)
      def _(b0):
        av = cidx_v[pl.ds(j * bpw + b0, LANES)]
        offv = jnp.where(av >= hc, DIM, 0)
        for kk in range(LANES):
          off = offv[kk]
          for g in range(ngrp):
            sl = pl.ds(pl.multiple_of((b0 + kk) * DIM + g * LANES, LANES),
                       LANES)
            v = cbuf_v[0, b0 + kk,
                       pl.ds(pl.multiple_of(off + g * LANES, LANES), LANES)]
            if j == 0:
              cacc_v[sl] = v
            else:
              cacc_v[sl] = cacc_v[sl] + v

    pltpu.sync_copy(
        cacc_v,
        cats_hbm.at[pl.ds(pl.multiple_of(base * DIM, SEQP), bpw * DIM)])

    @pl.loop(0, nrow // NBUF)
    def _(i):
      for s in range(NBUF):
        slot = s
        other = 1 - s

        def stage_next():
          enc_desc(other, 0).wait()
          prep_and_fire(other)

        if s == 0:
          stage_next()
        else:
          pl.when(i < nrow // NBUF - 1)(stage_next)

        for d in gather_descs(slot):
          d.wait()

        zeros = (jnp.zeros((LANES,), jnp.float32),) * ngrp

        def tok_block(t0, carry, cnt):
          tv = enc_v[pl.ds(slot * SEQP + t0, LANES)]
          offv = jnp.where(tv >= hemb, DIM, 0)
          acc = list(carry)
          for kk in range(cnt):
            off = offv[kk]
            for g in range(ngrp):
              acc[g] = acc[g] + rows_v[
                  slot, t0 + kk,
                  pl.ds(pl.multiple_of(off + g * LANES, LANES), LANES)]
          return tuple(acc)

        @pl.loop(0, seq - seq % LANES, step=LANES, init_carry=zeros,
                 unroll=2)
        def totals(t0, carry):
          return tok_block(t0, carry, LANES)

        if seq % LANES:
          totals = tok_block(seq - seq % LANES, totals, seq % LANES)

        b_local = i * NBUF + s
        for g in range(ngrp):
          acc_v[pl.ds(pl.multiple_of(b_local * DIM + g * LANES, LANES),
                      LANES)] = totals[g]

        def refill():
          enc_desc(slot, base + b_local + NBUF).start()
        pl.when(i < nrow // NBUF - 1)(refill)

    pltpu.sync_copy(
        acc_v,
        sums_hbm.at[pl.ds(pl.multiple_of(base * DIM, SEQP), bpw * DIM)])

  return k(enc_flat, embp, c0p, c1p, c2p, cidx_flat)


def _tc_head(sums, cats, encoded_text, w_t, bias):
  bsz, seq = encoded_text.shape
  dim = sums.shape[1]
  ncls = w_t.shape[1]
  blk = 256

  def body(sums_ref, cats_ref, enc_ref, wt_ref, b_ref, out_ref):
    cnt = jnp.sum((enc_ref[...] != 0).astype(jnp.float32), axis=1,
                  keepdims=True)
    x = jnp.where(cnt > 0.0, sums_ref[...] / cnt, 0.0)
    x = x + cats_ref[...]
    z = lax.dot_general(x, wt_ref[...], (((1,), (0,)), ((), ())),
                        preferred_element_type=jnp.float32)
    out_ref[...] = z + b_ref[...]

  return pl.pallas_call(
      body,
      grid=(bsz // blk,),
      in_specs=[
          pl.BlockSpec((blk, dim), lambda i: (i, 0)),
          pl.BlockSpec((blk, dim), lambda i: (i, 0)),
          pl.BlockSpec((blk, seq), lambda i: (i, 0)),
          pl.BlockSpec((dim, ncls), lambda i: (0, 0)),
          pl.BlockSpec((1, ncls), lambda i: (0, 0)),
      ],
      out_specs=pl.BlockSpec((blk, ncls), lambda i: (i, 0)),
      out_shape=jax.ShapeDtypeStruct((bsz, ncls), jnp.float32),
  )(sums, cats, encoded_text, w_t, bias)


def kernel(encoded_text, additional_inputs, emb_table, cat0, cat1, cat2, W, b):
  bsz, seq = encoded_text.shape
  enc_flat = jnp.pad(encoded_text, ((0, 0), (0, SEQP - seq))).reshape(-1)
  cidx_flat = additional_inputs.T.reshape(-1)
  embp = _pack_table(emb_table)
  c0p, c1p, c2p = _pack_table(cat0), _pack_table(cat1), _pack_table(cat2)
  hcats = (cat0.shape[0] // 2, cat1.shape[0] // 2, cat2.shape[0] // 2)
  sums_f, cats_f = _sc_pool(enc_flat, embp, c0p, c1p, c2p, cidx_flat,
                            bsz, seq, emb_table.shape[0] // 2, hcats)
  sums = sums_f.reshape(bsz, DIM)
  cats = cats_f.reshape(bsz, DIM)
  return _tc_head(sums, cats, encoded_text, W.T, b.reshape(1, -1))


# speedup vs baseline: 1.4463x; 1.4396x over previous
import functools

import jax
import jax.numpy as jnp
from jax import lax
from jax.experimental import pallas as pl
from jax.experimental.pallas import tpu as pltpu
from jax.experimental.pallas import tpu_sc as plsc

LANES = 16
NWORKERS = 32
NBUF = 4
CHUNK = 128
DIM = 64
SEQP = 256


def _sc_pool(enc_flat, emb_table, cat0, cat1, cat2, add_flat, bsz, seq):
  bpw = bsz // NWORKERS
  ngrp = DIM // LANES
  mesh = plsc.VectorSubcoreMesh(core_axis_name="c", subcore_axis_name="s")

  @functools.partial(
      pl.kernel,
      out_type=(
          jax.ShapeDtypeStruct((bsz, DIM), jnp.float32),
          jax.ShapeDtypeStruct((bsz, DIM), jnp.float32),
      ),
      mesh=mesh,
      scratch_types=[
          pltpu.VMEM((NBUF * SEQP,), jnp.int32),
          pltpu.VMEM((NBUF, seq, DIM), jnp.float32),
          pltpu.VMEM((bpw, DIM), jnp.float32),
          pltpu.VMEM((3 * bpw,), jnp.int32),
          pltpu.VMEM((bpw, DIM), jnp.float32),
          pltpu.VMEM((bpw, DIM), jnp.float32),
          pltpu.VMEM((bpw, DIM), jnp.float32),
          pltpu.SemaphoreType.DMA,
          pltpu.SemaphoreType.DMA,
          pltpu.SemaphoreType.DMA,
          pltpu.SemaphoreType.DMA,
          pltpu.SemaphoreType.DMA,
          pltpu.SemaphoreType.DMA,
          pltpu.SemaphoreType.DMA,
          pltpu.SemaphoreType.DMA,
      ],
      compiler_params=pltpu.CompilerParams(use_tc_tiling_on_sc=False),
  )
  def k(enc_hbm, emb_hbm, c0_hbm, c1_hbm, c2_hbm, addt_hbm,
        sums_hbm, cats_hbm,
        enc_v, rows_v, acc_v, cidx_v, ca_v, cb_v, cc_v,
        se0, se1, se2, se3, sg0, sg1, sg2, sg3):
    sems_e = (se0, se1, se2, se3)
    sems_g = (sg0, sg1, sg2, sg3)
    wid = lax.axis_index("s") * 2 + lax.axis_index("c")
    base = wid * bpw

    def enc_desc(slot, row):
      return pltpu.make_async_copy(
          enc_hbm.at[pl.ds(row * SEQP, SEQP)],
          enc_v.at[pl.ds(slot * SEQP, SEQP)], sems_e[slot])

    def gather_descs(slot):
      return (
          pltpu.make_async_copy(
              emb_hbm.at[enc_v.at[pl.ds(slot * SEQP, CHUNK)]],
              rows_v.at[slot, pl.ds(0, CHUNK)], sems_g[slot]),
          pltpu.make_async_copy(
              emb_hbm.at[enc_v.at[pl.ds(slot * SEQP + CHUNK, seq - CHUNK)]],
              rows_v.at[slot, pl.ds(CHUNK, seq - CHUNK)], sems_g[slot]),
      )

    def fire(slot, row):
      enc_desc(slot, row).wait()
      for d in gather_descs(slot):
        d.start()

    for s in range(NBUF):
      enc_desc(s, base + s).start()
    fire(0, base)
    fire(1, base + 1)

    for j, (tab, dst) in enumerate(
        ((c0_hbm, ca_v), (c1_hbm, cb_v), (c2_hbm, cc_v))):
      pltpu.sync_copy(addt_hbm.at[pl.ds(j * bsz + base, bpw)],
                      cidx_v.at[pl.ds(j * bpw, bpw)])
      pltpu.sync_copy(tab.at[cidx_v.at[pl.ds(j * bpw, bpw)]], dst)

    @pl.loop(0, bpw, unroll=4)
    def _(b):
      for g in range(ngrp):
        sl = pl.ds(g * LANES, LANES)
        ca_v[b, sl] = ca_v[b, sl] + cb_v[b, sl] + cc_v[b, sl]

    pltpu.sync_copy(ca_v, cats_hbm.at[pl.ds(base, bpw)])

    @pl.loop(0, bpw // NBUF)
    def _(i):
      for s in range(NBUF):
        b_local = i * NBUF + s

        def stage():
          fire((s + 2) % NBUF, base + b_local + 2)
        if s < 2:
          stage()
        else:
          pl.when(i < bpw // NBUF - 1)(stage)

        for d in gather_descs(s):
          d.wait()

        zeros = (jnp.zeros((LANES,), jnp.float32),) * ngrp

        @pl.loop(0, seq, init_carry=zeros, unroll=8)
        def totals(t, carry):
          return tuple(
              c + rows_v[s, t, pl.ds(g * LANES, LANES)]
              for g, c in enumerate(carry))

        for g in range(ngrp):
          acc_v[b_local, pl.ds(g * LANES, LANES)] = totals[g]

        def refill():
          enc_desc(s, base + b_local + NBUF).start()
        pl.when(i < bpw // NBUF - 1)(refill)

    pltpu.sync_copy(acc_v, sums_hbm.at[pl.ds(base, bpw)])

  return k(enc_flat, emb_table, cat0, cat1, cat2, add_flat)


def _tc_head(sums, cats, encoded_text, w_t, bias):
  bsz, seq = encoded_text.shape
  dim = sums.shape[1]
  ncls = w_t.shape[1]
  blk = 256

  def body(sums_ref, cats_ref, enc_ref, wt_ref, b_ref, out_ref):
    cnt = jnp.sum((enc_ref[...] != 0).astype(jnp.float32), axis=1,
                  keepdims=True)
    x = jnp.where(cnt > 0.0, sums_ref[...] / cnt, 0.0)
    x = x + cats_ref[...]
    z = lax.dot_general(x, wt_ref[...], (((1,), (0,)), ((), ())),
                        preferred_element_type=jnp.float32)
    out_ref[...] = z + b_ref[...]

  return pl.pallas_call(
      body,
      grid=(bsz // blk,),
      in_specs=[
          pl.BlockSpec((blk, dim), lambda i: (i, 0)),
          pl.BlockSpec((blk, dim), lambda i: (i, 0)),
          pl.BlockSpec((blk, seq), lambda i: (i, 0)),
          pl.BlockSpec((dim, ncls), lambda i: (0, 0)),
          pl.BlockSpec((1, ncls), lambda i: (0, 0)),
      ],
      out_specs=pl.BlockSpec((blk, ncls), lambda i: (i, 0)),
      out_shape=jax.ShapeDtypeStruct((bsz, ncls), jnp.float32),
  )(sums, cats, encoded_text, w_t, bias)


def kernel(encoded_text, additional_inputs, emb_table, cat0, cat1, cat2, W, b):
  bsz, seq = encoded_text.shape
  enc_flat = jnp.pad(encoded_text, ((0, 0), (0, SEQP - seq))).reshape(-1)
  add_flat = additional_inputs.T.reshape(-1)
  sums, cats = _sc_pool(enc_flat, emb_table, cat0, cat1, cat2, add_flat,
                        bsz, seq)
  return _tc_head(sums, cats, encoded_text, W.T, b.reshape(1, -1))
